# Initial kernel scaffold; baseline (speedup 1.0000x reference)
import jax
import jax.numpy as jnp
from jax.experimental import pallas as pl


def kernel(item_inputs, table):
    raise NotImplementedError("write your pallas kernel here")


# speedup vs baseline: 4.6616x; 4.6616x over previous
import functools

import jax
import jax.numpy as jnp
from jax import lax
from jax.experimental import pallas as pl
from jax.experimental.pallas import tpu as pltpu
from jax.experimental.pallas import tpu_sc as plsc

D = 64
BATCH, SEQ = 4096, 50
B = BATCH * SEQ
SUB = 128
N_ROWS = B // SUB
NC, NS = 2, 16
NW = NC * NS
ROWS_PER_TILE = N_ROWS // NW
K = 10
N_CHUNK = ROWS_PER_TILE // K
CHUNK = K * SUB


def _build():
  mesh = plsc.VectorSubcoreMesh(core_axis_name="c", subcore_axis_name="s")

  @functools.partial(
      pl.kernel,
      mesh=mesh,
      compiler_params=pltpu.CompilerParams(use_tc_tiling_on_sc=False),
      out_type=jax.ShapeDtypeStruct((B, D), jnp.float32),
      scratch_types=[
          pltpu.VMEM((ROWS_PER_TILE, SUB), jnp.int32),
          pltpu.VMEM((CHUNK, D), jnp.float32),
          pltpu.SemaphoreType.DMA,
      ],
  )
  def emb(idx_hbm, table_hbm, out_hbm, idx_v, rows_v, sem):
    wid = lax.axis_index("s") * NC + lax.axis_index("c")
    pltpu.sync_copy(idx_hbm.at[wid], idx_v)
    base = pl.multiple_of(wid * (ROWS_PER_TILE * SUB), CHUNK)

    def chunk(c, carry):
      copies = [
          pltpu.async_copy(
              table_hbm.at[idx_v.at[c * K + k]],
              rows_v.at[pl.ds(k * SUB, SUB)],
              sem,
          )
          for k in range(K)
      ]
      for cp in copies:
        cp.wait()
      off = pl.multiple_of(base + c * CHUNK, CHUNK)
      pltpu.sync_copy(rows_v, out_hbm.at[pl.ds(off, CHUNK)])
      return carry

    lax.fori_loop(0, N_CHUNK, chunk, 0)

  return emb


_emb = _build()


def kernel(item_inputs, table):
  idx = item_inputs.astype(jnp.int32).reshape(NW, ROWS_PER_TILE, SUB)
  out = _emb(idx, table)
  return out.reshape(BATCH, SEQ, D)
